# baseline (speedup 1.0000x reference)
import functools

import jax
import jax.numpy as jnp
from jax import lax
from jax.experimental import pallas as pl
from jax.experimental.pallas import tpu as pltpu
from jax.experimental.pallas import tpu_sc as plsc

N = 10000
E = 320000
D = 128
H = 128
C = 40

NC = 2
NS = 16
L = 16
R = 632
NP = NS * R
EH = E // NC
K = 800
NCHUNK = EH // K
assert K % L == 0 and EH % K == 0


SHIFT = 16384
G = 256
GH = G // 2
CAP = 8224


def _sc_segmax_body(m_hbm, src_hbm, dst_hbm, out_hbm, agg_v, dbufA, sbufA,
                    dbufB, sbufB, dense, idxA, idxB, rowsA, semA, esemA,
                    esemB):
    c = lax.axis_index("c")
    s = lax.axis_index("s")
    lo = s * R
    base = c * EH

    zero16 = jnp.zeros((L,), jnp.float32)

    def zrow(r, carry):
        for j in range(H // L):
            agg_v[r, pl.ds(L * j, L)] = zero16
        return carry

    lax.fori_loop(0, R + 1, zrow, 0)

    def fire_edges(k, dbuf_r, sbuf_r, esem_r):
        off = base + k * K
        pltpu.async_copy(dst_hbm.at[pl.ds(off, K)], dbuf_r, esem_r)
        pltpu.async_copy(src_hbm.at[pl.ds(off, K)], sbuf_r, esem_r)

    def wait_edges(dbuf_r, sbuf_r, esem_r):
        pltpu.make_async_copy(dst_hbm.at[pl.ds(base, K)], dbuf_r,
                              esem_r).wait()
        pltpu.make_async_copy(src_hbm.at[pl.ds(base, K)], sbuf_r,
                              esem_r).wait()

    TRASH = jnp.full((L,), R * SHIFT, jnp.int32)

    def fire(off):
        for u in range(GH // L):
            pv = dense[pl.ds(off + L * u, L)]
            idxA[pl.ds(L * u, L)] = jnp.bitwise_and(pv, SHIFT - 1)
        for u in range(GH // L):
            pv = dense[pl.ds(off + GH + L * u, L)]
            idxB[pl.ds(L * u, L)] = jnp.bitwise_and(pv, SHIFT - 1)
        pltpu.async_copy(m_hbm.at[idxA], rowsA.at[pl.ds(0, GH)], semA)
        pltpu.async_copy(m_hbm.at[idxB], rowsA.at[pl.ds(GH, GH)], semA)

    def wait_acc(off):
        def acc_u(u, carry3):
            pv = dense[pl.ds(off + L * u, L)]
            dvec = lax.div(pv, SHIFT)
            for i in range(L):
                drow = dvec[i]
                rv = [rowsA[u * L + i, pl.ds(L * j, L)]
                      for j in range(H // L)]
                av = [agg_v[drow, pl.ds(L * j, L)] for j in range(H // L)]
                for j in range(H // L):
                    agg_v[drow, pl.ds(L * j, L)] = jnp.maximum(av[j], rv[j])
            return carry3

        pltpu.make_async_copy(m_hbm.at[idxA], rowsA.at[pl.ds(0, GH)],
                              semA).wait()
        lax.fori_loop(0, GH // L, acc_u, 0)
        pltpu.make_async_copy(m_hbm.at[idxB], rowsA.at[pl.ds(GH, GH)],
                              semA).wait()
        lax.fori_loop(GH // L, G // L, acc_u, 0)

    def process_chunk(dbuf_r, sbuf_r, st):
        cnt0, fh0, wh0 = st

        def compact(i, cnt):
            d = dbuf_r[pl.ds(L * i, L)]
            sv = sbuf_r[pl.ds(L * i, L)]
            dl = d - lo
            msk = (dl >= 0) & (dl < R)
            packed = dl * SHIFT + sv
            plsc.store_compressed(dense.at[pl.ds(cnt, L)], packed, mask=msk)
            pc = plsc.all_reduce_population_count(msk)
            return cnt + pc[0]

        cnt = lax.fori_loop(0, K // L, compact, cnt0)

        def fire_step(i, st2):
            fh2, wh2 = st2

            @pl.when(fh2 > wh2)
            def _consume():
                wait_acc(wh2)

            wh3 = jnp.where(fh2 > wh2, wh2 + G, wh2)
            fire(fh2)
            return fh2 + G, wh3

        fh, wh = lax.fori_loop(0, (cnt - fh0) // G, fire_step, (fh0, wh0))

        def flush(args):
            cnt4, fh4, wh4 = args

            @pl.when(fh4 > wh4)
            def _consume4():
                wait_acc(wh4)

            for u in range(G // L):
                v = dense[pl.ds(fh4 + L * u, L)]
                dense[pl.ds(L * u, L)] = v
            return cnt4 - fh4, jnp.int32(0), jnp.int32(0)

        return lax.cond(cnt + K + L > CAP, flush, lambda a: a,
                        (cnt, fh, wh))

    fire_edges(0, dbufA, sbufA, esemA)
    st0 = (jnp.int32(0), jnp.int32(0), jnp.int32(0))

    def chunk_pair(t, st):
        k0 = 2 * t
        wait_edges(dbufA, sbufA, esemA)
        fire_edges(k0 + 1, dbufB, sbufB, esemB)
        st = process_chunk(dbufA, sbufA, st)

        @pl.when(k0 + 2 < NCHUNK)
        def _next_a():
            fire_edges(k0 + 2, dbufA, sbufA, esemA)

        wait_edges(dbufB, sbufB, esemB)
        st = process_chunk(dbufB, sbufB, st)
        return st

    cnt, fh, wh = lax.fori_loop(0, NCHUNK // 2, chunk_pair, st0)

    @pl.when(fh > wh)
    def _final_pending():
        wait_acc(wh)

    for u in range(G // L):
        dense[pl.ds(cnt + L * u, L)] = TRASH

    @pl.when(cnt > fh)
    def _final_rem():
        fire(fh)
        wait_acc(fh)

    pltpu.sync_copy(agg_v.at[pl.ds(0, R)], out_hbm.at[c, pl.ds(lo, R)])


@functools.cache
def _sc_segmax_kernel():
    return pl.kernel(
        _sc_segmax_body,
        out_type=jax.ShapeDtypeStruct((NC, NP, H), jnp.float32),
        mesh=plsc.VectorSubcoreMesh(core_axis_name="c", subcore_axis_name="s",
                                    num_cores=NC, num_subcores=NS),
        compiler_params=pltpu.CompilerParams(needs_layout_passes=False),
        scratch_types=[
            pltpu.VMEM((R + 1, H), jnp.float32),
            pltpu.VMEM((K,), jnp.int32),
            pltpu.VMEM((K,), jnp.int32),
            pltpu.VMEM((K,), jnp.int32),
            pltpu.VMEM((K,), jnp.int32),
            pltpu.VMEM((CAP + G,), jnp.int32),
            pltpu.VMEM((GH,), jnp.int32),
            pltpu.VMEM((GH,), jnp.int32),
            pltpu.VMEM((G, H), jnp.float32),
            pltpu.SemaphoreType.DMA,
            pltpu.SemaphoreType.DMA,
            pltpu.SemaphoreType.DMA,
        ],
    )


def _sc_segmax(m, src, dst):
    return _sc_segmax_kernel()(m, src, dst)[:, :N]


BLK = 1000


def _tc_pool_body(x_ref, wp_ref, bp_ref, m_ref):
    m_ref[...] = jnp.maximum(
        jnp.dot(x_ref[...], wp_ref[...],
                preferred_element_type=jnp.float32) + bp_ref[...], 0.0)


def _tc_pool(x, W_pool, b_pool):
    return pl.pallas_call(
        _tc_pool_body,
        grid=(N // BLK,),
        in_specs=[
            pl.BlockSpec((BLK, D), lambda i: (i, 0)),
            pl.BlockSpec((D, H), lambda i: (0, 0)),
            pl.BlockSpec((H,), lambda i: (0,)),
        ],
        out_specs=pl.BlockSpec((BLK, H), lambda i: (i, 0)),
        out_shape=jax.ShapeDtypeStruct((N, H), jnp.float32),
    )(x, W_pool, b_pool)


def _tc_combine_body(h_ref, parts_ref, w1_ref, b1_ref, wp_ref, bp_ref,
                     h1_ref, m2_ref):
    agg = jnp.maximum(parts_ref[0], parts_ref[1])
    hcat = jnp.concatenate([h_ref[...], agg], axis=1)
    h1 = jnp.maximum(
        jnp.dot(hcat, w1_ref[...],
                preferred_element_type=jnp.float32) + b1_ref[...], 0.0)
    h1_ref[...] = h1
    m2_ref[...] = jnp.maximum(
        jnp.dot(h1, wp_ref[...],
                preferred_element_type=jnp.float32) + bp_ref[...], 0.0)


def _tc_combine(x, parts, W1, b1, W_pool, b_pool):
    return pl.pallas_call(
        _tc_combine_body,
        grid=(N // BLK,),
        in_specs=[
            pl.BlockSpec((BLK, D), lambda i: (i, 0)),
            pl.BlockSpec((NC, BLK, H), lambda i: (0, i, 0)),
            pl.BlockSpec((D + H, H), lambda i: (0, 0)),
            pl.BlockSpec((H,), lambda i: (0,)),
            pl.BlockSpec((H, H), lambda i: (0, 0)),
            pl.BlockSpec((H,), lambda i: (0,)),
        ],
        out_specs=[
            pl.BlockSpec((BLK, H), lambda i: (i, 0)),
            pl.BlockSpec((BLK, H), lambda i: (i, 0)),
        ],
        out_shape=[
            jax.ShapeDtypeStruct((N, H), jnp.float32),
            jax.ShapeDtypeStruct((N, H), jnp.float32),
        ],
    )(x, parts, W1, b1, W_pool, b_pool)


def _tc_final_body(h_ref, parts_ref, w2_ref, b2_ref, o_ref):
    agg = jnp.maximum(parts_ref[0], parts_ref[1])
    hcat = jnp.concatenate([h_ref[...], agg], axis=1)
    z = jnp.dot(hcat, w2_ref[...],
                preferred_element_type=jnp.float32) + b2_ref[...]
    zmax = jnp.max(z, axis=1, keepdims=True)
    zs = z - zmax
    lse = jnp.log(jnp.sum(jnp.exp(zs), axis=1, keepdims=True))
    o_ref[...] = zs - lse


def _tc_final(h1, parts, W2, b2):
    return pl.pallas_call(
        _tc_final_body,
        grid=(N // BLK,),
        in_specs=[
            pl.BlockSpec((BLK, H), lambda i: (i, 0)),
            pl.BlockSpec((NC, BLK, H), lambda i: (0, i, 0)),
            pl.BlockSpec((H + H, C), lambda i: (0, 0)),
            pl.BlockSpec((C,), lambda i: (0,)),
        ],
        out_specs=pl.BlockSpec((BLK, C), lambda i: (i, 0)),
        out_shape=jax.ShapeDtypeStruct((N, C), jnp.float32),
    )(h1, parts, W2, b2)


def kernel(x, edge_index, W_pool, b_pool, W1, b1, W2, b2):
    src = edge_index[0]
    dst = edge_index[1]
    m1 = _tc_pool(x, W_pool, b_pool)
    parts1 = _sc_segmax(m1, src, dst)
    h1, m2 = _tc_combine(x, parts1, W1, b1, W_pool, b_pool)
    parts2 = _sc_segmax(m2, src, dst)
    return _tc_final(h1, parts2, W2, b2)


# speedup vs baseline: 1.0087x; 1.0087x over previous
import functools

import jax
import jax.numpy as jnp
from jax import lax
from jax.experimental import pallas as pl
from jax.experimental.pallas import tpu as pltpu
from jax.experimental.pallas import tpu_sc as plsc

N = 10000
E = 320000
D = 128
H = 128
C = 40

NC = 2
NS = 16
L = 16
R = 632
NP = NS * R
EH = E // NC
K = 800
NCHUNK = EH // K
assert K % L == 0 and EH % K == 0


SHIFT = 16384
G = 256
GH = G // 2
CAP = 8224


def _sc_segmax_body(m_hbm, src_hbm, dst_hbm, out_hbm, agg_v, dbufA, sbufA,
                    dbufB, sbufB, dense, idxA, idxB, rowsA, semA, esemA,
                    esemB):
    c = lax.axis_index("c")
    s = lax.axis_index("s")
    lo = s * R
    base = c * EH

    zero16 = jnp.zeros((L,), jnp.float32)

    def zrow(r, carry):
        for j in range(H // L):
            agg_v[r, pl.ds(L * j, L)] = zero16
        return carry

    lax.fori_loop(0, R + 1, zrow, 0)

    def fire_edges(k, dbuf_r, sbuf_r, esem_r):
        off = base + k * K
        pltpu.async_copy(dst_hbm.at[pl.ds(off, K)], dbuf_r, esem_r)
        pltpu.async_copy(src_hbm.at[pl.ds(off, K)], sbuf_r, esem_r)

    def wait_edges(dbuf_r, sbuf_r, esem_r):
        pltpu.make_async_copy(dst_hbm.at[pl.ds(base, K)], dbuf_r,
                              esem_r).wait()
        pltpu.make_async_copy(src_hbm.at[pl.ds(base, K)], sbuf_r,
                              esem_r).wait()

    TRASH = jnp.full((L,), R * SHIFT, jnp.int32)

    def fire(off):
        for u in range(GH // L):
            pv = dense[pl.ds(off + L * u, L)]
            idxA[pl.ds(L * u, L)] = jnp.bitwise_and(pv, SHIFT - 1)
        for u in range(GH // L):
            pv = dense[pl.ds(off + GH + L * u, L)]
            idxB[pl.ds(L * u, L)] = jnp.bitwise_and(pv, SHIFT - 1)
        pltpu.async_copy(m_hbm.at[idxA], rowsA.at[pl.ds(0, GH)], semA)
        pltpu.async_copy(m_hbm.at[idxB], rowsA.at[pl.ds(GH, GH)], semA)

    def wait_acc(off):
        def acc_u(u, carry3):
            pv = dense[pl.ds(off + L * u, L)]
            dvec = lax.div(pv, SHIFT)
            for i in range(L):
                drow = dvec[i]
                rv = [rowsA[u * L + i, pl.ds(L * j, L)]
                      for j in range(H // L)]
                av = [agg_v[drow, pl.ds(L * j, L)] for j in range(H // L)]
                for j in range(H // L):
                    agg_v[drow, pl.ds(L * j, L)] = jnp.maximum(av[j], rv[j])
            return carry3

        pltpu.make_async_copy(m_hbm.at[idxA], rowsA.at[pl.ds(0, GH)],
                              semA).wait()
        pltpu.make_async_copy(m_hbm.at[idxB], rowsA.at[pl.ds(GH, GH)],
                              semA).wait()
        lax.fori_loop(0, G // L, acc_u, 0)

    def process_chunk(dbuf_r, sbuf_r, st):
        cnt0, fh0, wh0 = st

        def compact(i, cnt):
            d = dbuf_r[pl.ds(L * i, L)]
            sv = sbuf_r[pl.ds(L * i, L)]
            dl = d - lo
            msk = (dl >= 0) & (dl < R)
            packed = dl * SHIFT + sv
            plsc.store_compressed(dense.at[pl.ds(cnt, L)], packed, mask=msk)
            pc = plsc.all_reduce_population_count(msk)
            return cnt + pc[0]

        cnt = lax.fori_loop(0, K // L, compact, cnt0)

        def fire_step(i, st2):
            fh2, wh2 = st2

            @pl.when(fh2 > wh2)
            def _consume():
                wait_acc(wh2)

            wh3 = jnp.where(fh2 > wh2, wh2 + G, wh2)
            fire(fh2)
            return fh2 + G, wh3

        fh, wh = lax.fori_loop(0, (cnt - fh0) // G, fire_step, (fh0, wh0))

        def flush(args):
            cnt4, fh4, wh4 = args

            @pl.when(fh4 > wh4)
            def _consume4():
                wait_acc(wh4)

            for u in range(G // L):
                v = dense[pl.ds(fh4 + L * u, L)]
                dense[pl.ds(L * u, L)] = v
            return cnt4 - fh4, jnp.int32(0), jnp.int32(0)

        return lax.cond(cnt + K + L > CAP, flush, lambda a: a,
                        (cnt, fh, wh))

    fire_edges(0, dbufA, sbufA, esemA)
    st0 = (jnp.int32(0), jnp.int32(0), jnp.int32(0))

    def chunk_pair(t, st):
        k0 = 2 * t
        wait_edges(dbufA, sbufA, esemA)
        fire_edges(k0 + 1, dbufB, sbufB, esemB)
        st = process_chunk(dbufA, sbufA, st)

        @pl.when(k0 + 2 < NCHUNK)
        def _next_a():
            fire_edges(k0 + 2, dbufA, sbufA, esemA)

        wait_edges(dbufB, sbufB, esemB)
        st = process_chunk(dbufB, sbufB, st)
        return st

    cnt, fh, wh = lax.fori_loop(0, NCHUNK // 2, chunk_pair, st0)

    @pl.when(fh > wh)
    def _final_pending():
        wait_acc(wh)

    for u in range(G // L):
        dense[pl.ds(cnt + L * u, L)] = TRASH

    @pl.when(cnt > fh)
    def _final_rem():
        fire(fh)
        wait_acc(fh)

    pltpu.sync_copy(agg_v.at[pl.ds(0, R)], out_hbm.at[c, pl.ds(lo, R)])


@functools.cache
def _sc_segmax_kernel():
    return pl.kernel(
        _sc_segmax_body,
        out_type=jax.ShapeDtypeStruct((NC, NP, H), jnp.float32),
        mesh=plsc.VectorSubcoreMesh(core_axis_name="c", subcore_axis_name="s",
                                    num_cores=NC, num_subcores=NS),
        compiler_params=pltpu.CompilerParams(needs_layout_passes=False),
        scratch_types=[
            pltpu.VMEM((R + 1, H), jnp.float32),
            pltpu.VMEM((K,), jnp.int32),
            pltpu.VMEM((K,), jnp.int32),
            pltpu.VMEM((K,), jnp.int32),
            pltpu.VMEM((K,), jnp.int32),
            pltpu.VMEM((CAP + G,), jnp.int32),
            pltpu.VMEM((GH,), jnp.int32),
            pltpu.VMEM((GH,), jnp.int32),
            pltpu.VMEM((G, H), jnp.float32),
            pltpu.SemaphoreType.DMA,
            pltpu.SemaphoreType.DMA,
            pltpu.SemaphoreType.DMA,
        ],
    )


def _sc_segmax(m, src, dst):
    return _sc_segmax_kernel()(m, src, dst)[:, :N]


BLK = 1000


def _tc_pool_body(x_ref, wp_ref, bp_ref, m_ref):
    m_ref[...] = jnp.maximum(
        jnp.dot(x_ref[...], wp_ref[...],
                preferred_element_type=jnp.float32) + bp_ref[...], 0.0)


def _tc_pool(x, W_pool, b_pool):
    return pl.pallas_call(
        _tc_pool_body,
        grid=(N // BLK,),
        in_specs=[
            pl.BlockSpec((BLK, D), lambda i: (i, 0)),
            pl.BlockSpec((D, H), lambda i: (0, 0)),
            pl.BlockSpec((H,), lambda i: (0,)),
        ],
        out_specs=pl.BlockSpec((BLK, H), lambda i: (i, 0)),
        out_shape=jax.ShapeDtypeStruct((N, H), jnp.float32),
    )(x, W_pool, b_pool)


def _tc_combine_body(h_ref, parts_ref, w1_ref, b1_ref, wp_ref, bp_ref,
                     h1_ref, m2_ref):
    agg = jnp.maximum(parts_ref[0], parts_ref[1])
    hcat = jnp.concatenate([h_ref[...], agg], axis=1)
    h1 = jnp.maximum(
        jnp.dot(hcat, w1_ref[...],
                preferred_element_type=jnp.float32) + b1_ref[...], 0.0)
    h1_ref[...] = h1
    m2_ref[...] = jnp.maximum(
        jnp.dot(h1, wp_ref[...],
                preferred_element_type=jnp.float32) + bp_ref[...], 0.0)


def _tc_combine(x, parts, W1, b1, W_pool, b_pool):
    return pl.pallas_call(
        _tc_combine_body,
        grid=(N // BLK,),
        in_specs=[
            pl.BlockSpec((BLK, D), lambda i: (i, 0)),
            pl.BlockSpec((NC, BLK, H), lambda i: (0, i, 0)),
            pl.BlockSpec((D + H, H), lambda i: (0, 0)),
            pl.BlockSpec((H,), lambda i: (0,)),
            pl.BlockSpec((H, H), lambda i: (0, 0)),
            pl.BlockSpec((H,), lambda i: (0,)),
        ],
        out_specs=[
            pl.BlockSpec((BLK, H), lambda i: (i, 0)),
            pl.BlockSpec((BLK, H), lambda i: (i, 0)),
        ],
        out_shape=[
            jax.ShapeDtypeStruct((N, H), jnp.float32),
            jax.ShapeDtypeStruct((N, H), jnp.float32),
        ],
    )(x, parts, W1, b1, W_pool, b_pool)


def _tc_final_body(h_ref, parts_ref, w2_ref, b2_ref, o_ref):
    agg = jnp.maximum(parts_ref[0], parts_ref[1])
    hcat = jnp.concatenate([h_ref[...], agg], axis=1)
    z = jnp.dot(hcat, w2_ref[...],
                preferred_element_type=jnp.float32) + b2_ref[...]
    zmax = jnp.max(z, axis=1, keepdims=True)
    zs = z - zmax
    lse = jnp.log(jnp.sum(jnp.exp(zs), axis=1, keepdims=True))
    o_ref[...] = zs - lse


def _tc_final(h1, parts, W2, b2):
    return pl.pallas_call(
        _tc_final_body,
        grid=(N // BLK,),
        in_specs=[
            pl.BlockSpec((BLK, H), lambda i: (i, 0)),
            pl.BlockSpec((NC, BLK, H), lambda i: (0, i, 0)),
            pl.BlockSpec((H + H, C), lambda i: (0, 0)),
            pl.BlockSpec((C,), lambda i: (0,)),
        ],
        out_specs=pl.BlockSpec((BLK, C), lambda i: (i, 0)),
        out_shape=jax.ShapeDtypeStruct((N, C), jnp.float32),
    )(h1, parts, W2, b2)


def kernel(x, edge_index, W_pool, b_pool, W1, b1, W2, b2):
    src = edge_index[0]
    dst = edge_index[1]
    m1 = _tc_pool(x, W_pool, b_pool)
    parts1 = _sc_segmax(m1, src, dst)
    h1, m2 = _tc_combine(x, parts1, W1, b1, W_pool, b_pool)
    parts2 = _sc_segmax(m2, src, dst)
    return _tc_final(h1, parts2, W2, b2)
